# Initial kernel scaffold; baseline (speedup 1.0000x reference)
import jax
import jax.numpy as jnp
from jax.experimental import pallas as pl


def kernel(inputs, table):
    raise NotImplementedError("write your pallas kernel here")


# speedup vs baseline: 8.0653x; 8.0653x over previous
import jax
import jax.numpy as jnp
from jax.experimental import pallas as pl


def _body(inp_ref, unary_ref, binary_ref):
    x = inp_ref[...]
    nb, l = x.shape
    c = unary_ref.shape[2]
    xi = x[:, :, None]
    row = jax.lax.broadcasted_iota(jnp.int32, (nb, l, c), 1)
    col = jax.lax.broadcasted_iota(jnp.int32, (nb, l, c), 2)
    unary_ref[...] = (col == row).astype(jnp.float32) + (xi == col - l).astype(
        jnp.float32
    )
    xj = x[:, None, :]
    eq_a = (xi == xj[:, :, : l - 1]).astype(jnp.float32)
    eq_b = (xi == xj[:, :, 1:]).astype(jnp.float32)
    kk = jax.lax.broadcasted_iota(jnp.int32, (nb, l, l - 1), 2)
    ii = jax.lax.broadcasted_iota(jnp.int32, (nb, l, l - 1), 1)
    binary_ref[...] = jnp.where(kk < ii, eq_a, eq_b)


def kernel(inputs, table):
    bs, ls = inputs.shape
    c = ls + table.shape[0]
    nb = 8
    unary, binary = pl.pallas_call(
        _body,
        grid=(bs // nb,),
        in_specs=[pl.BlockSpec((nb, ls), lambda n: (n, 0))],
        out_specs=[
            pl.BlockSpec((nb, ls, c), lambda n: (n, 0, 0)),
            pl.BlockSpec((nb, ls, ls - 1), lambda n: (n, 0, 0)),
        ],
        out_shape=[
            jax.ShapeDtypeStruct((bs, ls, c), jnp.float32),
            jax.ShapeDtypeStruct((bs, ls, ls - 1), jnp.float32),
        ],
    )(inputs)
    return unary, binary[..., None]


# speedup vs baseline: 32.9130x; 4.0808x over previous
import jax
import jax.numpy as jnp
from jax.experimental import pallas as pl


def _unary_body(xt_ref, u_ref):
    cb, l, b = u_ref.shape
    c0 = pl.program_id(0) * cb
    cc = c0 + jax.lax.broadcasted_iota(jnp.int32, (cb, l, b), 0)
    ii = jax.lax.broadcasted_iota(jnp.int32, (cb, l, b), 1)
    xt = xt_ref[...][None]
    u_ref[...] = (cc == ii).astype(jnp.float32) + (xt == cc - l).astype(
        jnp.float32
    )


def _binary_body(xt_ref, xi_ref, b_ref):
    ib, lm1, bh, bl = b_ref.shape
    l = lm1 + 1
    i0 = pl.program_id(0) * ib
    xt = xt_ref[...]
    xi = xi_ref[...][:, None]
    eq_a = (xt[None, : l - 1] == xi).astype(jnp.float32)
    eq_b = (xt[None, 1:] == xi).astype(jnp.float32)
    kk = jax.lax.broadcasted_iota(jnp.int32, (ib, lm1, bh, bl), 1)
    ii = i0 + jax.lax.broadcasted_iota(jnp.int32, (ib, lm1, bh, bl), 0)
    b_ref[...] = jnp.where(kk < ii, eq_a, eq_b)


def kernel(inputs, table):
    bs, ls = inputs.shape
    c = ls + table.shape[0]
    xt = inputs.T
    cb = 7
    u_t = pl.pallas_call(
        _unary_body,
        grid=(c // cb,),
        in_specs=[pl.BlockSpec((ls, bs), lambda n: (0, 0))],
        out_specs=pl.BlockSpec((cb, ls, bs), lambda n: (n, 0, 0)),
        out_shape=jax.ShapeDtypeStruct((c, ls, bs), jnp.float32),
    )(xt)
    unary = jnp.transpose(u_t, (2, 1, 0))
    bh = bs // 128
    xt3 = xt.reshape(ls, bh, 128)
    ib = 8
    b4 = pl.pallas_call(
        _binary_body,
        grid=(ls // ib,),
        in_specs=[
            pl.BlockSpec((ls, bh, 128), lambda n: (0, 0, 0)),
            pl.BlockSpec((ib, bh, 128), lambda n: (n, 0, 0)),
        ],
        out_specs=pl.BlockSpec((ib, ls - 1, bh, 128), lambda n: (n, 0, 0, 0)),
        out_shape=jax.ShapeDtypeStruct((ls, ls - 1, bh, 128), jnp.float32),
    )(xt3, xt3)
    binary = jnp.transpose(b4, (2, 3, 0, 1)).reshape(bs, ls, ls - 1, 1)
    return unary, binary


# speedup vs baseline: 33.2345x; 1.0098x over previous
import jax
import jax.numpy as jnp
from jax.experimental import pallas as pl

_CB = 7
_IB = 8
_NU = 43


def _body(xt_ref, xt3_ref, xi_ref, u_ref, b_ref):
    n = pl.program_id(0)
    cb, l, b = u_ref.shape
    ib, lm1, bh, bl = b_ref.shape

    @pl.when(n < _NU)
    def _unary():
        c0 = n * cb
        cc = c0 + jax.lax.broadcasted_iota(jnp.int32, (cb, l, b), 0)
        ii = jax.lax.broadcasted_iota(jnp.int32, (cb, l, b), 1)
        xt = xt_ref[...][None]
        u_ref[...] = (cc == ii).astype(jnp.float32) + (xt == cc - l).astype(
            jnp.float32
        )

    @pl.when(n >= _NU)
    def _binary():
        i0 = (n - _NU) * ib
        xt = xt3_ref[...]
        xi = xi_ref[...][:, None]
        eq_a = (xt[None, : l - 1] == xi).astype(jnp.float32)
        eq_b = (xt[None, 1:] == xi).astype(jnp.float32)
        kk = jax.lax.broadcasted_iota(jnp.int32, (ib, lm1, bh, bl), 1)
        ii = i0 + jax.lax.broadcasted_iota(jnp.int32, (ib, lm1, bh, bl), 0)
        b_ref[...] = jnp.where(kk < ii, eq_a, eq_b)


def kernel(inputs, table):
    bs, ls = inputs.shape
    c = ls + table.shape[0]
    xt = inputs.T
    bh = bs // 128
    xt3 = xt.reshape(ls, bh, 128)
    u_t, b4 = pl.pallas_call(
        _body,
        grid=(_NU + ls // _IB,),
        in_specs=[
            pl.BlockSpec((ls, bs), lambda n: (0, 0)),
            pl.BlockSpec((ls, bh, 128), lambda n: (0, 0, 0)),
            pl.BlockSpec(
                (_IB, bh, 128),
                lambda n: (jnp.clip(n - _NU, 0, ls // _IB - 1), 0, 0),
            ),
        ],
        out_specs=[
            pl.BlockSpec(
                (_CB, ls, bs), lambda n: (jnp.minimum(n, _NU - 1), 0, 0)
            ),
            pl.BlockSpec(
                (_IB, ls - 1, bh, 128),
                lambda n: (jnp.clip(n - _NU, 0, ls // _IB - 1), 0, 0, 0),
            ),
        ],
        out_shape=[
            jax.ShapeDtypeStruct((c, ls, bs), jnp.float32),
            jax.ShapeDtypeStruct((ls, ls - 1, bh, 128), jnp.float32),
        ],
    )(xt, xt3, xt3)
    unary = jnp.transpose(u_t, (2, 1, 0))
    binary = jnp.transpose(b4, (2, 3, 0, 1)).reshape(bs, ls, ls - 1, 1)
    return unary, binary
